# Initial kernel scaffold; baseline (speedup 1.0000x reference)
import jax
import jax.numpy as jnp
from jax.experimental import pallas as pl


def kernel(x, pe):
    raise NotImplementedError("write your pallas kernel here")


# speedup vs baseline: 1.6816x; 1.6816x over previous
import jax
import jax.numpy as jnp
from jax.experimental import pallas as pl

_SB = 512


def _add_kernel(x_ref, pe_ref, o_ref):
    o_ref[...] = x_ref[...] + pe_ref[...][None, :, :]


def kernel(x, pe):
    B, S, D = x.shape
    pe_s = pe[:S]
    grid = (S // _SB, B)
    return pl.pallas_call(
        _add_kernel,
        grid=grid,
        in_specs=[
            pl.BlockSpec((1, _SB, D), lambda i, j: (j, i, 0)),
            pl.BlockSpec((_SB, D), lambda i, j: (i, 0)),
        ],
        out_specs=pl.BlockSpec((1, _SB, D), lambda i, j: (j, i, 0)),
        out_shape=jax.ShapeDtypeStruct((B, S, D), x.dtype),
    )(x, pe_s)


# speedup vs baseline: 1.8503x; 1.1003x over previous
import jax
import jax.numpy as jnp
from jax.experimental import pallas as pl

_SB = 1024


def _add_kernel(x_ref, pe_ref, o_ref):
    o_ref[...] = x_ref[...] + pe_ref[...][None, :, :]


def kernel(x, pe):
    B, S, D = x.shape
    pe_s = pe[:S]
    grid = (S // _SB, B)
    return pl.pallas_call(
        _add_kernel,
        grid=grid,
        in_specs=[
            pl.BlockSpec((1, _SB, D), lambda i, j: (j, i, 0)),
            pl.BlockSpec((_SB, D), lambda i, j: (i, 0)),
        ],
        out_specs=pl.BlockSpec((1, _SB, D), lambda i, j: (j, i, 0)),
        out_shape=jax.ShapeDtypeStruct((B, S, D), x.dtype),
    )(x, pe_s)


# speedup vs baseline: 1.9462x; 1.0518x over previous
import jax
import jax.numpy as jnp
from jax.experimental import pallas as pl

_SB = 512


def _add_kernel(x_ref, pe_ref, o_ref):
    o_ref[...] = x_ref[...] + pe_ref[...][None, :, :]


def kernel(x, pe):
    B, S, D = x.shape
    pe_s = pe[:S]
    grid = (S // _SB,)
    return pl.pallas_call(
        _add_kernel,
        grid=grid,
        in_specs=[
            pl.BlockSpec((B, _SB, D), lambda i: (0, i, 0)),
            pl.BlockSpec((_SB, D), lambda i: (i, 0)),
        ],
        out_specs=pl.BlockSpec((B, _SB, D), lambda i: (0, i, 0)),
        out_shape=jax.ShapeDtypeStruct((B, S, D), x.dtype),
    )(x, pe_s)
